# baseline (speedup 1.0000x reference)
import functools

import jax
import jax.numpy as jnp
from jax import lax
from jax.experimental import pallas as pl
from jax.experimental.pallas import tpu as pltpu
from jax.experimental.pallas import tpu_sc as plsc

N = 10000
E = 320000
IN_NF = 128
H = 64
OUT_NF = 128
EF = 16

NW = 32
CHUNK = 128
BN = 512
BE = 2048
N_PAD = 10240
E_PAD = 327680
NSEG = 2
ESEG = E_PAD // NSEG
EPW = ESEG // NW
NCH = EPW // CHUNK
PW = 80
TW = 80


def _node_pre_body(nf_ref, c8_ref, Win_ref, bin_ref, We1_ref,
                   h_ref, tR_ref, tC_ref):
    h = jnp.dot(nf_ref[...], Win_ref[...],
                preferred_element_type=jnp.float32) + bin_ref[...]
    h_ref[...] = h
    hA = jnp.dot(h, We1_ref[0:H], preferred_element_type=jnp.float32)
    hB = jnp.dot(h, We1_ref[H:2 * H], preferred_element_type=jnp.float32)
    c8 = c8_ref[...]
    z = jnp.zeros((BN, TW - H - 8), jnp.float32)
    tR_ref[...] = jnp.concatenate([hA, c8, z], axis=1)
    tC_ref[...] = jnp.concatenate([hB, -c8, z], axis=1)


def _node_pre(nf_p, c8, W_in, b_in, We1):
    grid = (N_PAD // BN,)
    return pl.pallas_call(
        _node_pre_body,
        grid=grid,
        in_specs=[
            pl.BlockSpec((BN, IN_NF), lambda i: (i, 0)),
            pl.BlockSpec((BN, 8), lambda i: (i, 0)),
            pl.BlockSpec((IN_NF, H), lambda i: (0, 0)),
            pl.BlockSpec((1, H), lambda i: (0, 0)),
            pl.BlockSpec((2 * H + 1 + EF, H), lambda i: (0, 0)),
        ],
        out_specs=[
            pl.BlockSpec((BN, H), lambda i: (i, 0)),
            pl.BlockSpec((BN, TW), lambda i: (i, 0)),
            pl.BlockSpec((BN, TW), lambda i: (i, 0)),
        ],
        out_shape=[
            jax.ShapeDtypeStruct((N_PAD, H), jnp.float32),
            jax.ShapeDtypeStruct((N_PAD, TW), jnp.float32),
            jax.ShapeDtypeStruct((N_PAD, TW), jnp.float32),
        ],
    )(nf_p, c8, W_in, b_in.reshape(1, H), We1)


_MESH = plsc.VectorSubcoreMesh(core_axis_name="c", subcore_axis_name="s")


NBUF = 5
DEPTH = 4


@functools.partial(
    pl.kernel,
    out_type=jax.ShapeDtypeStruct((ESEG, TW), jnp.float32),
    mesh=_MESH,
    compiler_params=pltpu.CompilerParams(use_tc_tiling_on_sc=False),
    scratch_types=[
        pltpu.VMEM((NCH, CHUNK), jnp.int32),
        pltpu.VMEM((NCH, CHUNK), jnp.int32),
        pltpu.VMEM((NBUF, CHUNK, TW), jnp.float32),
        pltpu.VMEM((NBUF, CHUNK, TW), jnp.float32),
        pltpu.SemaphoreType.DMA((NBUF,)),
        pltpu.SemaphoreType.DMA((NBUF,)),
    ],
)
def _sc_gather(tR_hbm, tC_hbm, row_hbm, col_hbm, gS_hbm,
               row_v, col_v, bufR, bufC, semG, semW):
    c = lax.axis_index("c")
    s = lax.axis_index("s")
    wid = s * 2 + c
    pltpu.sync_copy(row_hbm.at[pl.ds(wid * NCH, NCH)], row_v)
    pltpu.sync_copy(col_hbm.at[pl.ds(wid * NCH, NCH)], col_v)

    def start_gather(j, b):
        pltpu.async_copy(tR_hbm.at[row_v.at[j]], bufR.at[b], semG.at[b])
        pltpu.async_copy(tC_hbm.at[col_v.at[j]], bufC.at[b], semG.at[b])

    def wait_gather(j, b):
        pltpu.make_async_copy(tR_hbm.at[row_v.at[j]], bufR.at[b],
                              semG.at[b]).wait()
        pltpu.make_async_copy(tC_hbm.at[col_v.at[j]], bufC.at[b],
                              semG.at[b]).wait()

    def wait_write(j, b):
        base = wid * EPW + j * CHUNK
        pltpu.make_async_copy(bufR.at[b], gS_hbm.at[pl.ds(base, CHUNK)],
                              semW.at[b]).wait()

    for b in range(DEPTH):
        start_gather(b, b)

    def body(jj, carry):
        for b in range(NBUF):
            j = jj * NBUF + b
            wait_gather(j, b)

            def add_row(r, carry2):
                for l in range(TW // 16):
                    bufR[b, r, pl.ds(l * 16, 16)] = (
                        bufR[b, r, pl.ds(l * 16, 16)]
                        + bufC[b, r, pl.ds(l * 16, 16)])
                return carry2

            lax.fori_loop(0, CHUNK, add_row, 0, unroll=2)
            base = wid * EPW + j * CHUNK
            pltpu.async_copy(bufR.at[b], gS_hbm.at[pl.ds(base, CHUNK)],
                             semW.at[b])
            ns = (b + DEPTH) % NBUF

            @pl.when(j + DEPTH < NCH)
            def _():
                @pl.when(j >= 1)
                def _():
                    wait_write(j - 1, ns)
                start_gather(j + DEPTH, ns)

        return carry

    lax.fori_loop(0, NCH // NBUF, body, 0)
    for b in range(NBUF):
        wait_write(NCH - NBUF + b, b)


def _edge_body(g_ref, ea_ref, We1e_ref, wrad_ref, be1_ref,
               We2_ref, be2_ref, Wc1_ref, bc1_ref, wc2_ref, o_ref):
    g = g_ref[...].astype(jnp.float32)
    cd = g[:, H:H + 3]
    radial = jnp.sum(cd * cd, axis=1, keepdims=True)
    pre = (g[:, 0:H] + radial * wrad_ref[...]
           + jnp.dot(ea_ref[...], We1e_ref[...],
                     preferred_element_type=jnp.float32)
           + be1_ref[...])
    m1 = jnp.maximum(pre, 0.0)
    m = jnp.maximum(jnp.dot(m1, We2_ref[...],
                            preferred_element_type=jnp.float32)
                    + be2_ref[...], 0.0)
    cc = jnp.maximum(jnp.dot(m, Wc1_ref[...],
                             preferred_element_type=jnp.float32)
                     + bc1_ref[...], 0.0)
    sca = jnp.sum(cc * wc2_ref[...], axis=1, keepdims=True)
    trans = cd * sca
    ones = jnp.ones((BE, 1), jnp.float32)
    z = jnp.zeros((BE, PW - H - 4), jnp.float32)
    o_ref[...] = jnp.concatenate([m, trans, ones, z], axis=1)


def _edge_mlp(gS, ea_p, We1, be1, We2, be2, Wc1, bc1, Wc2):
    grid = (ESEG // BE,)
    We1e = We1[2 * H + 1:]
    wrad = We1[2 * H:2 * H + 1]
    return pl.pallas_call(
        _edge_body,
        grid=grid,
        in_specs=[
            pl.BlockSpec((BE, TW), lambda i: (i, 0)),
            pl.BlockSpec((BE, EF), lambda i: (i, 0)),
            pl.BlockSpec((EF, H), lambda i: (0, 0)),
            pl.BlockSpec((1, H), lambda i: (0, 0)),
            pl.BlockSpec((1, H), lambda i: (0, 0)),
            pl.BlockSpec((H, H), lambda i: (0, 0)),
            pl.BlockSpec((1, H), lambda i: (0, 0)),
            pl.BlockSpec((H, H), lambda i: (0, 0)),
            pl.BlockSpec((1, H), lambda i: (0, 0)),
            pl.BlockSpec((1, H), lambda i: (0, 0)),
        ],
        out_specs=pl.BlockSpec((BE, PW), lambda i: (i, 0)),
        out_shape=jax.ShapeDtypeStruct((ESEG, PW), jnp.float32),
    )(gS, ea_p, We1e, wrad, be1.reshape(1, H), We2,
      be2.reshape(1, H), Wc1, bc1.reshape(1, H), Wc2.reshape(1, H))


@functools.partial(
    pl.kernel,
    out_type=jax.ShapeDtypeStruct((2, N_PAD, PW), jnp.float32),
    mesh=_MESH,
    compiler_params=pltpu.CompilerParams(use_tc_tiling_on_sc=False),
    scratch_types=[
        pltpu.VMEM((NCH, CHUNK), jnp.int32),
        pltpu.VMEM((2, CHUNK, PW), jnp.float32),
        pltpu.VMEM_SHARED((N_PAD, PW), jnp.float32),
        pltpu.SemaphoreType.DMA((2,)),
    ],
)
def _sc_scatter(data_hbm, row_hbm, zero_hbm, out_hbm,
                idx_v, buf, acc, semL):
    c = lax.axis_index("c")
    s = lax.axis_index("s")
    wid = s * 2 + c

    @pl.when(s == 0)
    def _init():
        pltpu.sync_copy(zero_hbm, acc)

    plsc.subcore_barrier()
    pltpu.sync_copy(row_hbm.at[pl.ds(wid * NCH, NCH)], idx_v)

    def run_seg(data_hbm, idx_off):
        def start_load(j, b):
            base = wid * EPW + j * CHUNK
            pltpu.async_copy(data_hbm.at[pl.ds(base, CHUNK)], buf.at[b],
                             semL.at[b])

        def wait_load(j, b):
            base = wid * EPW + j * CHUNK
            pltpu.make_async_copy(data_hbm.at[pl.ds(base, CHUNK)],
                                  buf.at[b], semL.at[b]).wait()

        start_load(0, 0)
        start_load(1, 1)

        def body(jj, carry):
            for b in range(2):
                j = jj * 2 + b
                wait_load(j, b)
                pltpu.sync_copy(buf.at[b], acc.at[idx_v.at[idx_off + j]],
                                add=True)

                @pl.when(j + 2 < NCH)
                def _():
                    start_load(j + 2, b)

            return carry

        lax.fori_loop(0, NCH // 2, body, 0)

    run_seg(data_hbm, 0)
    plsc.subcore_barrier()

    @pl.when(s == 0)
    def _out():
        pltpu.sync_copy(acc, out_hbm.at[c])


def _node_post_body(h_ref, c8_ref, pa_ref, pb_ref, Wn1_ref, bn1_ref,
                    Wn2_ref, bn2_ref, Wout_ref, bout_ref, o_ref, co_ref):
    agg = pa_ref[0] + pa_ref[1] + pb_ref[0] + pb_ref[1]
    aggm = agg[:, 0:H]
    tr = agg[:, H:H + 3]
    cnt = agg[:, H + 3:H + 4]
    c8 = c8_ref[...]
    co3 = c8[:, 0:3] + tr / jnp.maximum(cnt, 1.0)
    co_ref[...] = jnp.concatenate([co3, jnp.zeros((BN, 5), jnp.float32)],
                                  axis=1)
    h = h_ref[...]
    t = jnp.maximum(
        jnp.dot(h, Wn1_ref[0:H], preferred_element_type=jnp.float32)
        + jnp.dot(aggm, Wn1_ref[H:2 * H], preferred_element_type=jnp.float32)
        + bn1_ref[...], 0.0)
    h2 = h + jnp.dot(t, Wn2_ref[...],
                     preferred_element_type=jnp.float32) + bn2_ref[...]
    o_ref[...] = jnp.dot(h2, Wout_ref[...],
                         preferred_element_type=jnp.float32) + bout_ref[...]


def _node_post(h, c8, pa, pb, Wn1, bn1, Wn2, bn2, W_out, b_out):
    grid = (N_PAD // BN,)
    return pl.pallas_call(
        _node_post_body,
        grid=grid,
        in_specs=[
            pl.BlockSpec((BN, H), lambda i: (i, 0)),
            pl.BlockSpec((BN, 8), lambda i: (i, 0)),
            pl.BlockSpec((2, BN, PW), lambda i: (0, i, 0)),
            pl.BlockSpec((2, BN, PW), lambda i: (0, i, 0)),
            pl.BlockSpec((2 * H, H), lambda i: (0, 0)),
            pl.BlockSpec((1, H), lambda i: (0, 0)),
            pl.BlockSpec((H, H), lambda i: (0, 0)),
            pl.BlockSpec((1, H), lambda i: (0, 0)),
            pl.BlockSpec((H, OUT_NF), lambda i: (0, 0)),
            pl.BlockSpec((1, OUT_NF), lambda i: (0, 0)),
        ],
        out_specs=[
            pl.BlockSpec((BN, OUT_NF), lambda i: (i, 0)),
            pl.BlockSpec((BN, 8), lambda i: (i, 0)),
        ],
        out_shape=[
            jax.ShapeDtypeStruct((N_PAD, OUT_NF), jnp.float32),
            jax.ShapeDtypeStruct((N_PAD, 8), jnp.float32),
        ],
    )(h, c8, pa, pb, Wn1, bn1.reshape(1, H), Wn2, bn2.reshape(1, H),
      W_out, b_out.reshape(1, OUT_NF))


def kernel(node_feats, edge_index, edge_attr, coords, W_in, b_in, We1, be1,
           We2, be2, Wc1, bc1, Wc2, Wn1, bn1, Wn2, bn2, W_out, b_out):
    f32 = jnp.float32
    nf_p = jnp.pad(node_feats, ((0, N_PAD - N), (0, 0)))
    c8 = jnp.pad(coords.astype(f32), ((0, N_PAD - N), (0, 5)))
    row = edge_index[0]
    col = edge_index[1]
    row_p = jnp.pad(row, (0, E_PAD - E),
                    constant_values=N).reshape(E_PAD // CHUNK, CHUNK)
    col_p = jnp.pad(col, (0, E_PAD - E)).reshape(E_PAD // CHUNK, CHUNK)
    ea_p = jnp.pad(edge_attr, ((0, E_PAD - E), (0, 0)))

    h, tR, tC = _node_pre(nf_p, c8, W_in, b_in, We1)
    nseg_rows = NW * NCH
    gs = []
    for g in range(NSEG):
        gs.append(_sc_gather(tR, tC,
                             row_p[g * nseg_rows:(g + 1) * nseg_rows],
                             col_p[g * nseg_rows:(g + 1) * nseg_rows]))
    ps = []
    for g in range(NSEG):
        ps.append(_edge_mlp(gs[g], ea_p[g * ESEG:(g + 1) * ESEG],
                            We1, be1, We2, be2, Wc1, bc1, Wc2))
    zero = jnp.zeros((N_PAD, PW), f32)
    parts = []
    for g in range(NSEG):
        parts.append(_sc_scatter(ps[g],
                                 row_p[g * nseg_rows:(g + 1) * nseg_rows],
                                 zero))
    out, co8 = _node_post(h, c8, parts[0], parts[1],
                          Wn1, bn1, Wn2, bn2, W_out, b_out)
    return out[:N], co8[:N, :3]


# speedup vs baseline: 1.0565x; 1.0565x over previous
import functools

import jax
import jax.numpy as jnp
from jax import lax
from jax.experimental import pallas as pl
from jax.experimental.pallas import tpu as pltpu
from jax.experimental.pallas import tpu_sc as plsc

N = 10000
E = 320000
IN_NF = 128
H = 64
OUT_NF = 128
EF = 16

NW = 32
CHUNK = 128
BN = 512
BE = 2048
N_PAD = 10240
E_PAD = 327680
NSEG = 2
ESEG = E_PAD // NSEG
EPW = ESEG // NW
NCH = EPW // CHUNK
PW = 80
TW = 80


def _node_pre_body(nf_ref, c8_ref, Win_ref, bin_ref, We1_ref,
                   h_ref, tR_ref, tC_ref):
    h = jnp.dot(nf_ref[...], Win_ref[...],
                preferred_element_type=jnp.float32) + bin_ref[...]
    h_ref[...] = h
    hA = jnp.dot(h, We1_ref[0:H], preferred_element_type=jnp.float32)
    hB = jnp.dot(h, We1_ref[H:2 * H], preferred_element_type=jnp.float32)
    c8 = c8_ref[...]
    z = jnp.zeros((BN, TW - H - 8), jnp.float32)
    tR_ref[...] = jnp.concatenate([hA, c8, z], axis=1)
    tC_ref[...] = jnp.concatenate([hB, -c8, z], axis=1)


def _node_pre(nf_p, c8, W_in, b_in, We1):
    grid = (N_PAD // BN,)
    return pl.pallas_call(
        _node_pre_body,
        grid=grid,
        in_specs=[
            pl.BlockSpec((BN, IN_NF), lambda i: (i, 0)),
            pl.BlockSpec((BN, 8), lambda i: (i, 0)),
            pl.BlockSpec((IN_NF, H), lambda i: (0, 0)),
            pl.BlockSpec((1, H), lambda i: (0, 0)),
            pl.BlockSpec((2 * H + 1 + EF, H), lambda i: (0, 0)),
        ],
        out_specs=[
            pl.BlockSpec((BN, H), lambda i: (i, 0)),
            pl.BlockSpec((BN, TW), lambda i: (i, 0)),
            pl.BlockSpec((BN, TW), lambda i: (i, 0)),
        ],
        out_shape=[
            jax.ShapeDtypeStruct((N_PAD, H), jnp.float32),
            jax.ShapeDtypeStruct((N_PAD, TW), jnp.float32),
            jax.ShapeDtypeStruct((N_PAD, TW), jnp.float32),
        ],
    )(nf_p, c8, W_in, b_in.reshape(1, H), We1)


_MESH = plsc.VectorSubcoreMesh(core_axis_name="c", subcore_axis_name="s")


NBUF = 4
DEPTH = 3


@functools.partial(
    pl.kernel,
    out_type=jax.ShapeDtypeStruct((ESEG, TW), jnp.float32),
    mesh=_MESH,
    compiler_params=pltpu.CompilerParams(use_tc_tiling_on_sc=False),
    scratch_types=[
        pltpu.VMEM((NCH, CHUNK), jnp.int32),
        pltpu.VMEM((NCH, CHUNK), jnp.int32),
        pltpu.VMEM((NBUF, CHUNK, TW), jnp.float32),
        pltpu.VMEM((NBUF, CHUNK, TW), jnp.float32),
        pltpu.SemaphoreType.DMA((NBUF,)),
        pltpu.SemaphoreType.DMA((NBUF,)),
    ],
)
def _sc_gather(tR_hbm, tC_hbm, row_hbm, col_hbm, gS_hbm,
               row_v, col_v, bufR, bufC, semG, semW):
    c = lax.axis_index("c")
    s = lax.axis_index("s")
    wid = s * 2 + c
    pltpu.sync_copy(row_hbm.at[pl.ds(wid * NCH, NCH)], row_v)
    pltpu.sync_copy(col_hbm.at[pl.ds(wid * NCH, NCH)], col_v)

    def start_gather(j, b):
        pltpu.async_copy(tR_hbm.at[row_v.at[j]], bufR.at[b], semG.at[b])
        pltpu.async_copy(tC_hbm.at[col_v.at[j]], bufC.at[b], semG.at[b])

    def wait_gather(j, b):
        pltpu.make_async_copy(tR_hbm.at[row_v.at[j]], bufR.at[b],
                              semG.at[b]).wait()
        pltpu.make_async_copy(tC_hbm.at[col_v.at[j]], bufC.at[b],
                              semG.at[b]).wait()

    def wait_write(j, b):
        base = wid * EPW + j * CHUNK
        pltpu.make_async_copy(bufR.at[b], gS_hbm.at[pl.ds(base, CHUNK)],
                              semW.at[b]).wait()

    for b in range(DEPTH):
        start_gather(b, b)

    def body(jj, carry):
        for b in range(NBUF):
            j = jj * NBUF + b
            wait_gather(j, b)

            def add_row(r, carry2):
                for l in range(TW // 16):
                    bufR[b, r, pl.ds(l * 16, 16)] = (
                        bufR[b, r, pl.ds(l * 16, 16)]
                        + bufC[b, r, pl.ds(l * 16, 16)])
                return carry2

            lax.fori_loop(0, CHUNK, add_row, 0, unroll=2)
            base = wid * EPW + j * CHUNK
            pltpu.async_copy(bufR.at[b], gS_hbm.at[pl.ds(base, CHUNK)],
                             semW.at[b])
            ns = (b + DEPTH) % NBUF

            @pl.when(j + DEPTH < NCH)
            def _():
                @pl.when(j >= 1)
                def _():
                    wait_write(j - 1, ns)
                start_gather(j + DEPTH, ns)

        return carry

    lax.fori_loop(0, NCH // NBUF, body, 0)
    for b in range(NBUF):
        wait_write(NCH - NBUF + b, b)


def _edge_body(g_ref, ea_ref, We1e_ref, wrad_ref, be1_ref,
               We2_ref, be2_ref, Wc1_ref, bc1_ref, wc2_ref, o_ref):
    g = g_ref[...].astype(jnp.float32)
    cd = g[:, H:H + 3]
    radial = jnp.sum(cd * cd, axis=1, keepdims=True)
    pre = (g[:, 0:H] + radial * wrad_ref[...]
           + jnp.dot(ea_ref[...], We1e_ref[...],
                     preferred_element_type=jnp.float32)
           + be1_ref[...])
    m1 = jnp.maximum(pre, 0.0)
    m = jnp.maximum(jnp.dot(m1, We2_ref[...],
                            preferred_element_type=jnp.float32)
                    + be2_ref[...], 0.0)
    cc = jnp.maximum(jnp.dot(m, Wc1_ref[...],
                             preferred_element_type=jnp.float32)
                     + bc1_ref[...], 0.0)
    sca = jnp.sum(cc * wc2_ref[...], axis=1, keepdims=True)
    trans = cd * sca
    ones = jnp.ones((BE, 1), jnp.float32)
    z = jnp.zeros((BE, PW - H - 4), jnp.float32)
    o_ref[...] = jnp.concatenate([m, trans, ones, z], axis=1)


def _edge_mlp(gS, ea_p, We1, be1, We2, be2, Wc1, bc1, Wc2):
    grid = (ESEG // BE,)
    We1e = We1[2 * H + 1:]
    wrad = We1[2 * H:2 * H + 1]
    return pl.pallas_call(
        _edge_body,
        grid=grid,
        in_specs=[
            pl.BlockSpec((BE, TW), lambda i: (i, 0)),
            pl.BlockSpec((BE, EF), lambda i: (i, 0)),
            pl.BlockSpec((EF, H), lambda i: (0, 0)),
            pl.BlockSpec((1, H), lambda i: (0, 0)),
            pl.BlockSpec((1, H), lambda i: (0, 0)),
            pl.BlockSpec((H, H), lambda i: (0, 0)),
            pl.BlockSpec((1, H), lambda i: (0, 0)),
            pl.BlockSpec((H, H), lambda i: (0, 0)),
            pl.BlockSpec((1, H), lambda i: (0, 0)),
            pl.BlockSpec((1, H), lambda i: (0, 0)),
        ],
        out_specs=pl.BlockSpec((BE, PW), lambda i: (i, 0)),
        out_shape=jax.ShapeDtypeStruct((ESEG, PW), jnp.float32),
    )(gS, ea_p, We1e, wrad, be1.reshape(1, H), We2,
      be2.reshape(1, H), Wc1, bc1.reshape(1, H), Wc2.reshape(1, H))


@functools.partial(
    pl.kernel,
    out_type=jax.ShapeDtypeStruct((2, N_PAD, PW), jnp.float32),
    mesh=_MESH,
    compiler_params=pltpu.CompilerParams(use_tc_tiling_on_sc=False),
    scratch_types=[
        pltpu.VMEM((2 * NCH, CHUNK), jnp.int32),
        pltpu.VMEM((2, CHUNK, PW), jnp.float32),
        pltpu.VMEM_SHARED((N_PAD, PW), jnp.float32),
        pltpu.SemaphoreType.DMA((2,)),
    ],
)
def _sc_scatter(dataA_hbm, dataB_hbm, row_hbm, zero_hbm, out_hbm,
                idx_v, buf, acc, semL):
    c = lax.axis_index("c")
    s = lax.axis_index("s")
    wid = s * 2 + c

    @pl.when(s == 0)
    def _init():
        pltpu.sync_copy(zero_hbm, acc)

    plsc.subcore_barrier()
    nhalf = NW * NCH
    pltpu.sync_copy(row_hbm.at[pl.ds(wid * NCH, NCH)],
                    idx_v.at[pl.ds(0, NCH)])
    pltpu.sync_copy(row_hbm.at[pl.ds(nhalf + wid * NCH, NCH)],
                    idx_v.at[pl.ds(NCH, NCH)])

    def run_seg(data_hbm, idx_off):
        def start_load(j, b):
            base = wid * EPW + j * CHUNK
            pltpu.async_copy(data_hbm.at[pl.ds(base, CHUNK)], buf.at[b],
                             semL.at[b])

        def wait_load(j, b):
            base = wid * EPW + j * CHUNK
            pltpu.make_async_copy(data_hbm.at[pl.ds(base, CHUNK)],
                                  buf.at[b], semL.at[b]).wait()

        start_load(0, 0)
        start_load(1, 1)

        def body(jj, carry):
            for b in range(2):
                j = jj * 2 + b
                wait_load(j, b)
                pltpu.sync_copy(buf.at[b], acc.at[idx_v.at[idx_off + j]],
                                add=True)

                @pl.when(j + 2 < NCH)
                def _():
                    start_load(j + 2, b)

            return carry

        lax.fori_loop(0, NCH // 2, body, 0)

    run_seg(dataA_hbm, 0)
    run_seg(dataB_hbm, NCH)
    plsc.subcore_barrier()

    @pl.when(s == 0)
    def _out():
        pltpu.sync_copy(acc, out_hbm.at[c])


def _node_post_body(h_ref, c8_ref, p_ref, Wn1_ref, bn1_ref,
                    Wn2_ref, bn2_ref, Wout_ref, bout_ref, o_ref, co_ref):
    agg = p_ref[0] + p_ref[1]
    aggm = agg[:, 0:H]
    tr = agg[:, H:H + 3]
    cnt = agg[:, H + 3:H + 4]
    c8 = c8_ref[...]
    co3 = c8[:, 0:3] + tr / jnp.maximum(cnt, 1.0)
    co_ref[...] = jnp.concatenate([co3, jnp.zeros((BN, 5), jnp.float32)],
                                  axis=1)
    h = h_ref[...]
    t = jnp.maximum(
        jnp.dot(h, Wn1_ref[0:H], preferred_element_type=jnp.float32)
        + jnp.dot(aggm, Wn1_ref[H:2 * H], preferred_element_type=jnp.float32)
        + bn1_ref[...], 0.0)
    h2 = h + jnp.dot(t, Wn2_ref[...],
                     preferred_element_type=jnp.float32) + bn2_ref[...]
    o_ref[...] = jnp.dot(h2, Wout_ref[...],
                         preferred_element_type=jnp.float32) + bout_ref[...]


def _node_post(h, c8, partials, Wn1, bn1, Wn2, bn2, W_out, b_out):
    grid = (N_PAD // BN,)
    return pl.pallas_call(
        _node_post_body,
        grid=grid,
        in_specs=[
            pl.BlockSpec((BN, H), lambda i: (i, 0)),
            pl.BlockSpec((BN, 8), lambda i: (i, 0)),
            pl.BlockSpec((2, BN, PW), lambda i: (0, i, 0)),
            pl.BlockSpec((2 * H, H), lambda i: (0, 0)),
            pl.BlockSpec((1, H), lambda i: (0, 0)),
            pl.BlockSpec((H, H), lambda i: (0, 0)),
            pl.BlockSpec((1, H), lambda i: (0, 0)),
            pl.BlockSpec((H, OUT_NF), lambda i: (0, 0)),
            pl.BlockSpec((1, OUT_NF), lambda i: (0, 0)),
        ],
        out_specs=[
            pl.BlockSpec((BN, OUT_NF), lambda i: (i, 0)),
            pl.BlockSpec((BN, 8), lambda i: (i, 0)),
        ],
        out_shape=[
            jax.ShapeDtypeStruct((N_PAD, OUT_NF), jnp.float32),
            jax.ShapeDtypeStruct((N_PAD, 8), jnp.float32),
        ],
    )(h, c8, partials, Wn1, bn1.reshape(1, H), Wn2, bn2.reshape(1, H),
      W_out, b_out.reshape(1, OUT_NF))


def kernel(node_feats, edge_index, edge_attr, coords, W_in, b_in, We1, be1,
           We2, be2, Wc1, bc1, Wc2, Wn1, bn1, Wn2, bn2, W_out, b_out):
    f32 = jnp.float32
    nf_p = jnp.pad(node_feats, ((0, N_PAD - N), (0, 0)))
    c8 = jnp.pad(coords.astype(f32), ((0, N_PAD - N), (0, 5)))
    row = edge_index[0]
    col = edge_index[1]
    row_p = jnp.pad(row, (0, E_PAD - E),
                    constant_values=N).reshape(E_PAD // CHUNK, CHUNK)
    col_p = jnp.pad(col, (0, E_PAD - E)).reshape(E_PAD // CHUNK, CHUNK)
    ea_p = jnp.pad(edge_attr, ((0, E_PAD - E), (0, 0)))

    h, tR, tC = _node_pre(nf_p, c8, W_in, b_in, We1)
    nseg_rows = NW * NCH
    gs = []
    for g in range(NSEG):
        gs.append(_sc_gather(tR, tC,
                             row_p[g * nseg_rows:(g + 1) * nseg_rows],
                             col_p[g * nseg_rows:(g + 1) * nseg_rows]))
    ps = []
    for g in range(NSEG):
        ps.append(_edge_mlp(gs[g], ea_p[g * ESEG:(g + 1) * ESEG],
                            We1, be1, We2, be2, Wc1, bc1, Wc2))
    zero = jnp.zeros((N_PAD, PW), f32)
    partials = _sc_scatter(ps[0], ps[1], row_p, zero)
    out, co8 = _node_post(h, c8, partials,
                          Wn1, bn1, Wn2, bn2, W_out, b_out)
    return out[:N], co8[:N, :3]


# speedup vs baseline: 1.1022x; 1.0433x over previous
import functools

import jax
import jax.numpy as jnp
from jax import lax
from jax.experimental import pallas as pl
from jax.experimental.pallas import tpu as pltpu
from jax.experimental.pallas import tpu_sc as plsc

N = 10000
E = 320000
IN_NF = 128
H = 64
OUT_NF = 128
EF = 16

NW = 32
CHUNK = 128
BN = 512
BE = 4096
N_PAD = 10240
E_PAD = 327680
NSEG = 2
ESEG = E_PAD // NSEG
EPW = ESEG // NW
NCH = EPW // CHUNK
PW = 80
TW = 80


def _node_pre_body(nf_ref, c8_ref, Win_ref, bin_ref, We1_ref,
                   h_ref, tR_ref, tC_ref):
    h = jnp.dot(nf_ref[...], Win_ref[...],
                preferred_element_type=jnp.float32) + bin_ref[...]
    h_ref[...] = h
    hA = jnp.dot(h, We1_ref[0:H], preferred_element_type=jnp.float32)
    hB = jnp.dot(h, We1_ref[H:2 * H], preferred_element_type=jnp.float32)
    c8 = c8_ref[...]
    z = jnp.zeros((BN, TW - H - 8), jnp.float32)
    tR_ref[...] = jnp.concatenate([hA, c8, z], axis=1)
    tC_ref[...] = jnp.concatenate([hB, -c8, z], axis=1)


def _node_pre(nf_p, c8, W_in, b_in, We1):
    grid = (N_PAD // BN,)
    return pl.pallas_call(
        _node_pre_body,
        grid=grid,
        in_specs=[
            pl.BlockSpec((BN, IN_NF), lambda i: (i, 0)),
            pl.BlockSpec((BN, 8), lambda i: (i, 0)),
            pl.BlockSpec((IN_NF, H), lambda i: (0, 0)),
            pl.BlockSpec((1, H), lambda i: (0, 0)),
            pl.BlockSpec((2 * H + 1 + EF, H), lambda i: (0, 0)),
        ],
        out_specs=[
            pl.BlockSpec((BN, H), lambda i: (i, 0)),
            pl.BlockSpec((BN, TW), lambda i: (i, 0)),
            pl.BlockSpec((BN, TW), lambda i: (i, 0)),
        ],
        out_shape=[
            jax.ShapeDtypeStruct((N_PAD, H), jnp.float32),
            jax.ShapeDtypeStruct((N_PAD, TW), jnp.float32),
            jax.ShapeDtypeStruct((N_PAD, TW), jnp.float32),
        ],
    )(nf_p, c8, W_in, b_in.reshape(1, H), We1)


_MESH = plsc.VectorSubcoreMesh(core_axis_name="c", subcore_axis_name="s")


NBUF = 4
DEPTH = 3


@functools.partial(
    pl.kernel,
    out_type=jax.ShapeDtypeStruct((ESEG, TW), jnp.float32),
    mesh=_MESH,
    compiler_params=pltpu.CompilerParams(use_tc_tiling_on_sc=False),
    scratch_types=[
        pltpu.VMEM((NCH, CHUNK), jnp.int32),
        pltpu.VMEM((NCH, CHUNK), jnp.int32),
        pltpu.VMEM((NBUF, CHUNK, TW), jnp.float32),
        pltpu.VMEM((NBUF, CHUNK, TW), jnp.float32),
        pltpu.SemaphoreType.DMA((NBUF,)),
        pltpu.SemaphoreType.DMA((NBUF,)),
    ],
)
def _sc_gather(tR_hbm, tC_hbm, row_hbm, col_hbm, gS_hbm,
               row_v, col_v, bufR, bufC, semG, semW):
    c = lax.axis_index("c")
    s = lax.axis_index("s")
    wid = s * 2 + c
    pltpu.sync_copy(row_hbm.at[pl.ds(wid * NCH, NCH)], row_v)
    pltpu.sync_copy(col_hbm.at[pl.ds(wid * NCH, NCH)], col_v)

    def start_gather(j, b):
        pltpu.async_copy(tR_hbm.at[row_v.at[j]], bufR.at[b], semG.at[b])
        pltpu.async_copy(tC_hbm.at[col_v.at[j]], bufC.at[b], semG.at[b])

    def wait_gather(j, b):
        pltpu.make_async_copy(tR_hbm.at[row_v.at[j]], bufR.at[b],
                              semG.at[b]).wait()
        pltpu.make_async_copy(tC_hbm.at[col_v.at[j]], bufC.at[b],
                              semG.at[b]).wait()

    def wait_write(j, b):
        base = wid * EPW + j * CHUNK
        pltpu.make_async_copy(bufR.at[b], gS_hbm.at[pl.ds(base, CHUNK)],
                              semW.at[b]).wait()

    for b in range(DEPTH):
        start_gather(b, b)

    def body(jj, carry):
        for b in range(NBUF):
            j = jj * NBUF + b
            wait_gather(j, b)

            def add_row(r, carry2):
                for l in range(TW // 16):
                    bufR[b, r, pl.ds(l * 16, 16)] = (
                        bufR[b, r, pl.ds(l * 16, 16)]
                        + bufC[b, r, pl.ds(l * 16, 16)])
                return carry2

            lax.fori_loop(0, CHUNK, add_row, 0, unroll=2)
            base = wid * EPW + j * CHUNK
            pltpu.async_copy(bufR.at[b], gS_hbm.at[pl.ds(base, CHUNK)],
                             semW.at[b])
            ns = (b + DEPTH) % NBUF

            @pl.when(j + DEPTH < NCH)
            def _():
                @pl.when(j >= 1)
                def _():
                    wait_write(j - 1, ns)
                start_gather(j + DEPTH, ns)

        return carry

    lax.fori_loop(0, NCH // NBUF, body, 0)
    for b in range(NBUF):
        wait_write(NCH - NBUF + b, b)


def _edge_body(g_ref, ea_ref, We1e_ref, wrad_ref, be1_ref,
               We2_ref, be2_ref, Wc1_ref, bc1_ref, wc2_ref, o_ref):
    g = g_ref[...].astype(jnp.float32)
    cd = g[:, H:H + 3]
    radial = jnp.sum(cd * cd, axis=1, keepdims=True)
    pre = (g[:, 0:H] + radial * wrad_ref[...]
           + jnp.dot(ea_ref[...], We1e_ref[...],
                     preferred_element_type=jnp.float32)
           + be1_ref[...])
    m1 = jnp.maximum(pre, 0.0)
    m = jnp.maximum(jnp.dot(m1, We2_ref[...],
                            preferred_element_type=jnp.float32)
                    + be2_ref[...], 0.0)
    cc = jnp.maximum(jnp.dot(m, Wc1_ref[...],
                             preferred_element_type=jnp.float32)
                     + bc1_ref[...], 0.0)
    sca = jnp.sum(cc * wc2_ref[...], axis=1, keepdims=True)
    trans = cd * sca
    ones = jnp.ones((BE, 1), jnp.float32)
    z = jnp.zeros((BE, PW - H - 4), jnp.float32)
    o_ref[...] = jnp.concatenate([m, trans, ones, z], axis=1)


def _edge_mlp(gS, ea_p, We1, be1, We2, be2, Wc1, bc1, Wc2):
    grid = (ESEG // BE,)
    We1e = We1[2 * H + 1:]
    wrad = We1[2 * H:2 * H + 1]
    return pl.pallas_call(
        _edge_body,
        grid=grid,
        in_specs=[
            pl.BlockSpec((BE, TW), lambda i: (i, 0)),
            pl.BlockSpec((BE, EF), lambda i: (i, 0)),
            pl.BlockSpec((EF, H), lambda i: (0, 0)),
            pl.BlockSpec((1, H), lambda i: (0, 0)),
            pl.BlockSpec((1, H), lambda i: (0, 0)),
            pl.BlockSpec((H, H), lambda i: (0, 0)),
            pl.BlockSpec((1, H), lambda i: (0, 0)),
            pl.BlockSpec((H, H), lambda i: (0, 0)),
            pl.BlockSpec((1, H), lambda i: (0, 0)),
            pl.BlockSpec((1, H), lambda i: (0, 0)),
        ],
        out_specs=pl.BlockSpec((BE, PW), lambda i: (i, 0)),
        out_shape=jax.ShapeDtypeStruct((ESEG, PW), jnp.float32),
    )(gS, ea_p, We1e, wrad, be1.reshape(1, H), We2,
      be2.reshape(1, H), Wc1, bc1.reshape(1, H), Wc2.reshape(1, H))


@functools.partial(
    pl.kernel,
    out_type=jax.ShapeDtypeStruct((2, N_PAD, PW), jnp.float32),
    mesh=_MESH,
    compiler_params=pltpu.CompilerParams(use_tc_tiling_on_sc=False),
    scratch_types=[
        pltpu.VMEM((2 * NCH, CHUNK), jnp.int32),
        pltpu.VMEM((2, CHUNK, PW), jnp.float32),
        pltpu.VMEM_SHARED((N_PAD, PW), jnp.float32),
        pltpu.SemaphoreType.DMA((2,)),
    ],
)
def _sc_scatter(dataA_hbm, dataB_hbm, row_hbm, zero_hbm, out_hbm,
                idx_v, buf, acc, semL):
    c = lax.axis_index("c")
    s = lax.axis_index("s")
    wid = s * 2 + c

    @pl.when(s == 0)
    def _init():
        pltpu.sync_copy(zero_hbm, acc)

    plsc.subcore_barrier()
    nhalf = NW * NCH
    pltpu.sync_copy(row_hbm.at[pl.ds(wid * NCH, NCH)],
                    idx_v.at[pl.ds(0, NCH)])
    pltpu.sync_copy(row_hbm.at[pl.ds(nhalf + wid * NCH, NCH)],
                    idx_v.at[pl.ds(NCH, NCH)])

    def run_seg(data_hbm, idx_off):
        def start_load(j, b):
            base = wid * EPW + j * CHUNK
            pltpu.async_copy(data_hbm.at[pl.ds(base, CHUNK)], buf.at[b],
                             semL.at[b])

        def wait_load(j, b):
            base = wid * EPW + j * CHUNK
            pltpu.make_async_copy(data_hbm.at[pl.ds(base, CHUNK)],
                                  buf.at[b], semL.at[b]).wait()

        start_load(0, 0)
        start_load(1, 1)

        def body(jj, carry):
            for b in range(2):
                j = jj * 2 + b
                wait_load(j, b)
                pltpu.sync_copy(buf.at[b], acc.at[idx_v.at[idx_off + j]],
                                add=True)

                @pl.when(j + 2 < NCH)
                def _():
                    start_load(j + 2, b)

            return carry

        lax.fori_loop(0, NCH // 2, body, 0)

    run_seg(dataA_hbm, 0)
    run_seg(dataB_hbm, NCH)
    plsc.subcore_barrier()

    @pl.when(s == 0)
    def _out():
        pltpu.sync_copy(acc, out_hbm.at[c])


def _node_post_body(h_ref, c8_ref, p_ref, Wn1_ref, bn1_ref,
                    Wn2_ref, bn2_ref, Wout_ref, bout_ref, o_ref, co_ref):
    agg = p_ref[0] + p_ref[1]
    aggm = agg[:, 0:H]
    tr = agg[:, H:H + 3]
    cnt = agg[:, H + 3:H + 4]
    c8 = c8_ref[...]
    co3 = c8[:, 0:3] + tr / jnp.maximum(cnt, 1.0)
    co_ref[...] = jnp.concatenate([co3, jnp.zeros((BN, 5), jnp.float32)],
                                  axis=1)
    h = h_ref[...]
    t = jnp.maximum(
        jnp.dot(h, Wn1_ref[0:H], preferred_element_type=jnp.float32)
        + jnp.dot(aggm, Wn1_ref[H:2 * H], preferred_element_type=jnp.float32)
        + bn1_ref[...], 0.0)
    h2 = h + jnp.dot(t, Wn2_ref[...],
                     preferred_element_type=jnp.float32) + bn2_ref[...]
    o_ref[...] = jnp.dot(h2, Wout_ref[...],
                         preferred_element_type=jnp.float32) + bout_ref[...]


def _node_post(h, c8, partials, Wn1, bn1, Wn2, bn2, W_out, b_out):
    grid = (N_PAD // BN,)
    return pl.pallas_call(
        _node_post_body,
        grid=grid,
        in_specs=[
            pl.BlockSpec((BN, H), lambda i: (i, 0)),
            pl.BlockSpec((BN, 8), lambda i: (i, 0)),
            pl.BlockSpec((2, BN, PW), lambda i: (0, i, 0)),
            pl.BlockSpec((2 * H, H), lambda i: (0, 0)),
            pl.BlockSpec((1, H), lambda i: (0, 0)),
            pl.BlockSpec((H, H), lambda i: (0, 0)),
            pl.BlockSpec((1, H), lambda i: (0, 0)),
            pl.BlockSpec((H, OUT_NF), lambda i: (0, 0)),
            pl.BlockSpec((1, OUT_NF), lambda i: (0, 0)),
        ],
        out_specs=[
            pl.BlockSpec((BN, OUT_NF), lambda i: (i, 0)),
            pl.BlockSpec((BN, 8), lambda i: (i, 0)),
        ],
        out_shape=[
            jax.ShapeDtypeStruct((N_PAD, OUT_NF), jnp.float32),
            jax.ShapeDtypeStruct((N_PAD, 8), jnp.float32),
        ],
    )(h, c8, partials, Wn1, bn1.reshape(1, H), Wn2, bn2.reshape(1, H),
      W_out, b_out.reshape(1, OUT_NF))


def kernel(node_feats, edge_index, edge_attr, coords, W_in, b_in, We1, be1,
           We2, be2, Wc1, bc1, Wc2, Wn1, bn1, Wn2, bn2, W_out, b_out):
    f32 = jnp.float32
    nf_p = jnp.pad(node_feats, ((0, N_PAD - N), (0, 0)))
    c8 = jnp.pad(coords.astype(f32), ((0, N_PAD - N), (0, 5)))
    row = edge_index[0]
    col = edge_index[1]
    row_p = jnp.pad(row, (0, E_PAD - E),
                    constant_values=N).reshape(E_PAD // CHUNK, CHUNK)
    col_p = jnp.pad(col, (0, E_PAD - E)).reshape(E_PAD // CHUNK, CHUNK)
    ea_p = jnp.pad(edge_attr, ((0, E_PAD - E), (0, 0)))

    h, tR, tC = _node_pre(nf_p, c8, W_in, b_in, We1)
    nseg_rows = NW * NCH
    gs = []
    for g in range(NSEG):
        gs.append(_sc_gather(tR, tC,
                             row_p[g * nseg_rows:(g + 1) * nseg_rows],
                             col_p[g * nseg_rows:(g + 1) * nseg_rows]))
    ps = []
    for g in range(NSEG):
        ps.append(_edge_mlp(gs[g], ea_p[g * ESEG:(g + 1) * ESEG],
                            We1, be1, We2, be2, Wc1, bc1, Wc2))
    zero = jnp.zeros((N_PAD, PW), f32)
    partials = _sc_scatter(ps[0], ps[1], row_p, zero)
    out, co8 = _node_post(h, c8, partials,
                          Wn1, bn1, Wn2, bn2, W_out, b_out)
    return out[:N], co8[:N, :3]


# speedup vs baseline: 1.1193x; 1.0155x over previous
import functools

import jax
import jax.numpy as jnp
from jax import lax
from jax.experimental import pallas as pl
from jax.experimental.pallas import tpu as pltpu
from jax.experimental.pallas import tpu_sc as plsc

N = 10000
E = 320000
IN_NF = 128
H = 64
OUT_NF = 128
EF = 16

NW = 32
CHUNK = 128
BN = 512
BE = 8192
N_PAD = 10240
E_PAD = 327680
NSEG = 2
ESEG = E_PAD // NSEG
EPW = ESEG // NW
NCH = EPW // CHUNK
PW = 80
TW = 80


def _node_pre_body(nf_ref, c8_ref, Win_ref, bin_ref, We1_ref,
                   h_ref, tR_ref, tC_ref):
    h = jnp.dot(nf_ref[...], Win_ref[...],
                preferred_element_type=jnp.float32) + bin_ref[...]
    h_ref[...] = h
    hA = jnp.dot(h, We1_ref[0:H], preferred_element_type=jnp.float32)
    hB = jnp.dot(h, We1_ref[H:2 * H], preferred_element_type=jnp.float32)
    c8 = c8_ref[...]
    z = jnp.zeros((BN, TW - H - 8), jnp.float32)
    tR_ref[...] = jnp.concatenate([hA, c8, z], axis=1)
    tC_ref[...] = jnp.concatenate([hB, -c8, z], axis=1)


def _node_pre(nf_p, c8, W_in, b_in, We1):
    grid = (N_PAD // BN,)
    return pl.pallas_call(
        _node_pre_body,
        grid=grid,
        in_specs=[
            pl.BlockSpec((BN, IN_NF), lambda i: (i, 0)),
            pl.BlockSpec((BN, 8), lambda i: (i, 0)),
            pl.BlockSpec((IN_NF, H), lambda i: (0, 0)),
            pl.BlockSpec((1, H), lambda i: (0, 0)),
            pl.BlockSpec((2 * H + 1 + EF, H), lambda i: (0, 0)),
        ],
        out_specs=[
            pl.BlockSpec((BN, H), lambda i: (i, 0)),
            pl.BlockSpec((BN, TW), lambda i: (i, 0)),
            pl.BlockSpec((BN, TW), lambda i: (i, 0)),
        ],
        out_shape=[
            jax.ShapeDtypeStruct((N_PAD, H), jnp.float32),
            jax.ShapeDtypeStruct((N_PAD, TW), jnp.float32),
            jax.ShapeDtypeStruct((N_PAD, TW), jnp.float32),
        ],
    )(nf_p, c8, W_in, b_in.reshape(1, H), We1)


_MESH = plsc.VectorSubcoreMesh(core_axis_name="c", subcore_axis_name="s")


NBUF = 4
DEPTH = 3


@functools.partial(
    pl.kernel,
    out_type=jax.ShapeDtypeStruct((ESEG, TW), jnp.float32),
    mesh=_MESH,
    compiler_params=pltpu.CompilerParams(use_tc_tiling_on_sc=False),
    scratch_types=[
        pltpu.VMEM((NCH, CHUNK), jnp.int32),
        pltpu.VMEM((NCH, CHUNK), jnp.int32),
        pltpu.VMEM((NBUF, CHUNK, TW), jnp.float32),
        pltpu.VMEM((NBUF, CHUNK, TW), jnp.float32),
        pltpu.SemaphoreType.DMA((NBUF,)),
        pltpu.SemaphoreType.DMA((NBUF,)),
    ],
)
def _sc_gather(tR_hbm, tC_hbm, row_hbm, col_hbm, gS_hbm,
               row_v, col_v, bufR, bufC, semG, semW):
    c = lax.axis_index("c")
    s = lax.axis_index("s")
    wid = s * 2 + c
    pltpu.sync_copy(row_hbm.at[pl.ds(wid * NCH, NCH)], row_v)
    pltpu.sync_copy(col_hbm.at[pl.ds(wid * NCH, NCH)], col_v)

    def start_gather(j, b):
        pltpu.async_copy(tR_hbm.at[row_v.at[j]], bufR.at[b], semG.at[b])
        pltpu.async_copy(tC_hbm.at[col_v.at[j]], bufC.at[b], semG.at[b])

    def wait_gather(j, b):
        pltpu.make_async_copy(tR_hbm.at[row_v.at[j]], bufR.at[b],
                              semG.at[b]).wait()
        pltpu.make_async_copy(tC_hbm.at[col_v.at[j]], bufC.at[b],
                              semG.at[b]).wait()

    def wait_write(j, b):
        base = wid * EPW + j * CHUNK
        pltpu.make_async_copy(bufR.at[b], gS_hbm.at[pl.ds(base, CHUNK)],
                              semW.at[b]).wait()

    for b in range(DEPTH):
        start_gather(b, b)

    def body(jj, carry):
        for b in range(NBUF):
            j = jj * NBUF + b
            wait_gather(j, b)

            def add_row(r, carry2):
                for l in range(TW // 16):
                    bufR[b, r, pl.ds(l * 16, 16)] = (
                        bufR[b, r, pl.ds(l * 16, 16)]
                        + bufC[b, r, pl.ds(l * 16, 16)])
                return carry2

            lax.fori_loop(0, CHUNK, add_row, 0, unroll=2)
            base = wid * EPW + j * CHUNK
            pltpu.async_copy(bufR.at[b], gS_hbm.at[pl.ds(base, CHUNK)],
                             semW.at[b])
            ns = (b + DEPTH) % NBUF

            @pl.when(j + DEPTH < NCH)
            def _():
                @pl.when(j >= 1)
                def _():
                    wait_write(j - 1, ns)
                start_gather(j + DEPTH, ns)

        return carry

    lax.fori_loop(0, NCH // NBUF, body, 0)
    for b in range(NBUF):
        wait_write(NCH - NBUF + b, b)


def _edge_body(g_ref, ea_ref, We1e_ref, wrad_ref, be1_ref,
               We2_ref, be2_ref, Wc1_ref, bc1_ref, wc2_ref, o_ref):
    g = g_ref[...].astype(jnp.float32)
    cd = g[:, H:H + 3]
    radial = jnp.sum(cd * cd, axis=1, keepdims=True)
    pre = (g[:, 0:H] + radial * wrad_ref[...]
           + jnp.dot(ea_ref[...], We1e_ref[...],
                     preferred_element_type=jnp.float32)
           + be1_ref[...])
    m1 = jnp.maximum(pre, 0.0)
    m = jnp.maximum(jnp.dot(m1, We2_ref[...],
                            preferred_element_type=jnp.float32)
                    + be2_ref[...], 0.0)
    cc = jnp.maximum(jnp.dot(m, Wc1_ref[...],
                             preferred_element_type=jnp.float32)
                     + bc1_ref[...], 0.0)
    sca = jnp.sum(cc * wc2_ref[...], axis=1, keepdims=True)
    trans = cd * sca
    ones = jnp.ones((BE, 1), jnp.float32)
    z = jnp.zeros((BE, PW - H - 4), jnp.float32)
    o_ref[...] = jnp.concatenate([m, trans, ones, z], axis=1)


def _edge_mlp(gS, ea_p, We1, be1, We2, be2, Wc1, bc1, Wc2):
    grid = (ESEG // BE,)
    We1e = We1[2 * H + 1:]
    wrad = We1[2 * H:2 * H + 1]
    return pl.pallas_call(
        _edge_body,
        grid=grid,
        in_specs=[
            pl.BlockSpec((BE, TW), lambda i: (i, 0)),
            pl.BlockSpec((BE, EF), lambda i: (i, 0)),
            pl.BlockSpec((EF, H), lambda i: (0, 0)),
            pl.BlockSpec((1, H), lambda i: (0, 0)),
            pl.BlockSpec((1, H), lambda i: (0, 0)),
            pl.BlockSpec((H, H), lambda i: (0, 0)),
            pl.BlockSpec((1, H), lambda i: (0, 0)),
            pl.BlockSpec((H, H), lambda i: (0, 0)),
            pl.BlockSpec((1, H), lambda i: (0, 0)),
            pl.BlockSpec((1, H), lambda i: (0, 0)),
        ],
        out_specs=pl.BlockSpec((BE, PW), lambda i: (i, 0)),
        out_shape=jax.ShapeDtypeStruct((ESEG, PW), jnp.float32),
    )(gS, ea_p, We1e, wrad, be1.reshape(1, H), We2,
      be2.reshape(1, H), Wc1, bc1.reshape(1, H), Wc2.reshape(1, H))


@functools.partial(
    pl.kernel,
    out_type=jax.ShapeDtypeStruct((2, N_PAD, PW), jnp.float32),
    mesh=_MESH,
    compiler_params=pltpu.CompilerParams(use_tc_tiling_on_sc=False),
    scratch_types=[
        pltpu.VMEM((2 * NCH, CHUNK), jnp.int32),
        pltpu.VMEM((2, CHUNK, PW), jnp.float32),
        pltpu.VMEM_SHARED((N_PAD, PW), jnp.float32),
        pltpu.SemaphoreType.DMA((2,)),
    ],
)
def _sc_scatter(dataA_hbm, dataB_hbm, row_hbm, zero_hbm, out_hbm,
                idx_v, buf, acc, semL):
    c = lax.axis_index("c")
    s = lax.axis_index("s")
    wid = s * 2 + c

    @pl.when(s == 0)
    def _init():
        pltpu.sync_copy(zero_hbm, acc)

    plsc.subcore_barrier()
    nhalf = NW * NCH
    pltpu.sync_copy(row_hbm.at[pl.ds(wid * NCH, NCH)],
                    idx_v.at[pl.ds(0, NCH)])
    pltpu.sync_copy(row_hbm.at[pl.ds(nhalf + wid * NCH, NCH)],
                    idx_v.at[pl.ds(NCH, NCH)])

    def run_seg(data_hbm, idx_off):
        def start_load(j, b):
            base = wid * EPW + j * CHUNK
            pltpu.async_copy(data_hbm.at[pl.ds(base, CHUNK)], buf.at[b],
                             semL.at[b])

        def wait_load(j, b):
            base = wid * EPW + j * CHUNK
            pltpu.make_async_copy(data_hbm.at[pl.ds(base, CHUNK)],
                                  buf.at[b], semL.at[b]).wait()

        start_load(0, 0)
        start_load(1, 1)

        def body(jj, carry):
            for b in range(2):
                j = jj * 2 + b
                wait_load(j, b)
                pltpu.sync_copy(buf.at[b], acc.at[idx_v.at[idx_off + j]],
                                add=True)

                @pl.when(j + 2 < NCH)
                def _():
                    start_load(j + 2, b)

            return carry

        lax.fori_loop(0, NCH // 2, body, 0)

    run_seg(dataA_hbm, 0)
    run_seg(dataB_hbm, NCH)
    plsc.subcore_barrier()

    @pl.when(s == 0)
    def _out():
        pltpu.sync_copy(acc, out_hbm.at[c])


def _node_post_body(h_ref, c8_ref, p_ref, Wn1_ref, bn1_ref,
                    Wn2_ref, bn2_ref, Wout_ref, bout_ref, o_ref, co_ref):
    agg = p_ref[0] + p_ref[1]
    aggm = agg[:, 0:H]
    tr = agg[:, H:H + 3]
    cnt = agg[:, H + 3:H + 4]
    c8 = c8_ref[...]
    co3 = c8[:, 0:3] + tr / jnp.maximum(cnt, 1.0)
    co_ref[...] = jnp.concatenate([co3, jnp.zeros((BN, 5), jnp.float32)],
                                  axis=1)
    h = h_ref[...]
    t = jnp.maximum(
        jnp.dot(h, Wn1_ref[0:H], preferred_element_type=jnp.float32)
        + jnp.dot(aggm, Wn1_ref[H:2 * H], preferred_element_type=jnp.float32)
        + bn1_ref[...], 0.0)
    h2 = h + jnp.dot(t, Wn2_ref[...],
                     preferred_element_type=jnp.float32) + bn2_ref[...]
    o_ref[...] = jnp.dot(h2, Wout_ref[...],
                         preferred_element_type=jnp.float32) + bout_ref[...]


def _node_post(h, c8, partials, Wn1, bn1, Wn2, bn2, W_out, b_out):
    grid = (N_PAD // BN,)
    return pl.pallas_call(
        _node_post_body,
        grid=grid,
        in_specs=[
            pl.BlockSpec((BN, H), lambda i: (i, 0)),
            pl.BlockSpec((BN, 8), lambda i: (i, 0)),
            pl.BlockSpec((2, BN, PW), lambda i: (0, i, 0)),
            pl.BlockSpec((2 * H, H), lambda i: (0, 0)),
            pl.BlockSpec((1, H), lambda i: (0, 0)),
            pl.BlockSpec((H, H), lambda i: (0, 0)),
            pl.BlockSpec((1, H), lambda i: (0, 0)),
            pl.BlockSpec((H, OUT_NF), lambda i: (0, 0)),
            pl.BlockSpec((1, OUT_NF), lambda i: (0, 0)),
        ],
        out_specs=[
            pl.BlockSpec((BN, OUT_NF), lambda i: (i, 0)),
            pl.BlockSpec((BN, 8), lambda i: (i, 0)),
        ],
        out_shape=[
            jax.ShapeDtypeStruct((N_PAD, OUT_NF), jnp.float32),
            jax.ShapeDtypeStruct((N_PAD, 8), jnp.float32),
        ],
    )(h, c8, partials, Wn1, bn1.reshape(1, H), Wn2, bn2.reshape(1, H),
      W_out, b_out.reshape(1, OUT_NF))


def kernel(node_feats, edge_index, edge_attr, coords, W_in, b_in, We1, be1,
           We2, be2, Wc1, bc1, Wc2, Wn1, bn1, Wn2, bn2, W_out, b_out):
    f32 = jnp.float32
    nf_p = jnp.pad(node_feats, ((0, N_PAD - N), (0, 0)))
    c8 = jnp.pad(coords.astype(f32), ((0, N_PAD - N), (0, 5)))
    row = edge_index[0]
    col = edge_index[1]
    row_p = jnp.pad(row, (0, E_PAD - E),
                    constant_values=N).reshape(E_PAD // CHUNK, CHUNK)
    col_p = jnp.pad(col, (0, E_PAD - E)).reshape(E_PAD // CHUNK, CHUNK)
    ea_p = jnp.pad(edge_attr, ((0, E_PAD - E), (0, 0)))

    h, tR, tC = _node_pre(nf_p, c8, W_in, b_in, We1)
    nseg_rows = NW * NCH
    gs = []
    for g in range(NSEG):
        gs.append(_sc_gather(tR, tC,
                             row_p[g * nseg_rows:(g + 1) * nseg_rows],
                             col_p[g * nseg_rows:(g + 1) * nseg_rows]))
    ps = []
    for g in range(NSEG):
        ps.append(_edge_mlp(gs[g], ea_p[g * ESEG:(g + 1) * ESEG],
                            We1, be1, We2, be2, Wc1, bc1, Wc2))
    zero = jnp.zeros((N_PAD, PW), f32)
    partials = _sc_scatter(ps[0], ps[1], row_p, zero)
    out, co8 = _node_post(h, c8, partials,
                          Wn1, bn1, Wn2, bn2, W_out, b_out)
    return out[:N], co8[:N, :3]
